# Initial kernel scaffold; baseline (speedup 1.0000x reference)
import jax
import jax.numpy as jnp
from jax.experimental import pallas as pl


def kernel(raw, edge_index, edge_attr, batch, params):
    raise NotImplementedError("write your pallas kernel here")


# speedup vs baseline: 8.0240x; 8.0240x over previous
import functools

import jax
import jax.numpy as jnp
from jax import lax
from jax.experimental import pallas as pl
from jax.experimental.pallas import tpu as pltpu
from jax.experimental.pallas import tpu_sc as plsc

N = 10000
E = 320000
D = 128
ED = 16
G = 256
H = 128
OUT = 128

NC = 2
NS = 16
NW = NC * NS
EPW = E // NW
LN = 16
CH = 80
NCHK = EPW // CH
NPT = 624
NTAIL = N - NS * NPT

EB = 2000
NB = 2000

def _wid():
    return lax.axis_index("s") * NC + lax.axis_index("c")


@functools.cache
def _mesh():
    return plsc.VectorSubcoreMesh(
        core_axis_name="c", subcore_axis_name="s",
        num_cores=NC, num_subcores=NS)


@functools.cache
def _build_gather1():
  return functools.partial(
    pl.kernel,
    out_type=jax.ShapeDtypeStruct((E,), jnp.float32),
    mesh=_mesh(),
    compiler_params=pltpu.CompilerParams(needs_layout_passes=False),
    scratch_types=[
        pltpu.VMEM((N,), jnp.float32),
        pltpu.VMEM((EPW,), jnp.int32),
        pltpu.VMEM((EPW,), jnp.float32),
    ],
  )(_gather1_body)


def _sc_gather1(tab, idx):
    return _build_gather1()(tab, idx)


def _gather1_body(tab_h, idx_h, out_h, tab_v, idx_v, out_v):
    base = _wid() * EPW
    pltpu.sync_copy(tab_h, tab_v)
    pltpu.sync_copy(idx_h.at[pl.ds(base, EPW)], idx_v)

    def body(i, carry):
        sl = pl.ds(i * LN, LN)
        out_v[sl] = plsc.load_gather(tab_v, [idx_v[sl]])
        return carry

    lax.fori_loop(0, EPW // LN, body, 0)
    pltpu.sync_copy(out_v, out_h.at[pl.ds(base, EPW)])


@functools.cache
def _build_gather2():
  return functools.partial(
    pl.kernel,
    out_type=(jax.ShapeDtypeStruct((E,), jnp.float32),
              jax.ShapeDtypeStruct((E,), jnp.float32)),
    mesh=_mesh(),
    compiler_params=pltpu.CompilerParams(needs_layout_passes=False),
    scratch_types=[
        pltpu.VMEM((N,), jnp.float32),
        pltpu.VMEM((EPW,), jnp.int32),
        pltpu.VMEM((EPW,), jnp.float32),
    ],
  )(_gather2_body)


def _sc_gather2(ta, ia, tb, ib):
    return _build_gather2()(ta, ia, tb, ib)


def _gather2_body(ta_h, ia_h, tb_h, ib_h, oa_h, ob_h, tab_v, idx_v, out_v):
    base = _wid() * EPW

    for tab_h, idx_h, out_h in ((ta_h, ia_h, oa_h), (tb_h, ib_h, ob_h)):
        pltpu.sync_copy(tab_h, tab_v)
        pltpu.sync_copy(idx_h.at[pl.ds(base, EPW)], idx_v)

        def body(i, carry):
            sl = pl.ds(i * LN, LN)
            out_v[sl] = plsc.load_gather(tab_v, [idx_v[sl]])
            return carry

        lax.fori_loop(0, EPW // LN, body, 0)
        pltpu.sync_copy(out_v, out_h.at[pl.ds(base, EPW)])


@functools.cache
def _build_gather_rows():
  return functools.partial(
    pl.kernel,
    out_type=jax.ShapeDtypeStruct((E, H), jnp.float32),
    mesh=_mesh(),
    compiler_params=pltpu.CompilerParams(needs_layout_passes=False),
    scratch_types=[
        pltpu.VMEM((EPW,), jnp.int32),
        pltpu.VMEM((CH, H), jnp.float32),
        pltpu.SemaphoreType.DMA,
    ],
  )(_gather_rows_body)


def _sc_gather_rows(x, idx):
    return _build_gather_rows()(x, idx)


def _gather_rows_body(x_h, idx_h, out_h, idx_v, rows_v, sem):
    base = _wid() * EPW
    pltpu.sync_copy(idx_h.at[pl.ds(base, EPW)], idx_v)

    def chunk(k, carry):
        cp = pltpu.async_copy(x_h.at[idx_v.at[pl.ds(k * CH, CH)]], rows_v, sem)
        cp.wait()
        pltpu.sync_copy(rows_v, out_h.at[pl.ds(base + k * CH, CH)])
        return carry

    lax.fori_loop(0, NCHK, chunk, 0)


@functools.cache
def _build_scatter_rows():
  return functools.partial(
    pl.kernel,
    out_type=(jax.ShapeDtypeStruct((NC, N, H), jnp.float32),
              jax.ShapeDtypeStruct((NW * N,), jnp.float32)),
    mesh=_mesh(),
    compiler_params=pltpu.CompilerParams(needs_layout_passes=False),
    scratch_types=[
        pltpu.VMEM((EPW,), jnp.int32),
        pltpu.VMEM((EPW,), jnp.float32),
        pltpu.VMEM((CH, H), jnp.float32),
        pltpu.VMEM((N,), jnp.float32),
        pltpu.SemaphoreType.DMA,
        pltpu.VMEM_SHARED((N, H), jnp.float32),
    ],
  )(_scatter_rows_body)


def _sc_scatter_rows(msg, e, dstf, z):
    return _build_scatter_rows()(msg, e, dstf, z)


def _scatter_rows_body(msg_h, e_h, dstf_h, z_h, acc_h, cnt_h,
                       idxf_v, e_v, rows_v, cnt_v, sem, shacc):
    c = lax.axis_index("c")
    s = lax.axis_index("s")
    w = s * NC + c
    base = w * EPW

    pltpu.sync_copy(dstf_h.at[pl.ds(base, EPW)], idxf_v)
    pltpu.sync_copy(e_h.at[pl.ds(base, EPW)], e_v)

    def zero(i, carry):
        cnt_v[pl.ds(i * LN, LN)] = jnp.zeros((LN,), jnp.float32)
        return carry

    lax.fori_loop(0, N // LN, zero, 0)
    off = pl.multiple_of(s * NPT, 8)
    pltpu.sync_copy(z_h.at[pl.ds(0, NPT)], shacc.at[pl.ds(off, NPT)])

    @pl.when(s == 0)
    def _zero_tail():
        pltpu.sync_copy(z_h.at[pl.ds(0, NTAIL)],
                        shacc.at[pl.ds(NS * NPT, NTAIL)])

    plsc.subcore_barrier()

    def chunk(k, carry):
        cp = pltpu.async_copy(msg_h.at[pl.ds(base + k * CH, CH)], rows_v, sem)
        cp.wait()
        pltpu.sync_copy(rows_v, shacc.at[idxf_v.at[pl.ds(k * CH, CH)]],
                        add=True)
        for j in range(CH // LN):
            sl = pl.ds(k * CH + j * LN, LN)
            plsc.addupdate_scatter(cnt_v, [idxf_v[sl]], e_v[sl])
        return carry

    lax.fori_loop(0, NCHK, chunk, 0)
    plsc.subcore_barrier()
    pltpu.sync_copy(shacc.at[pl.ds(off, NPT)],
                    acc_h.at[c, pl.ds(off, NPT)])

    @pl.when(s == 0)
    def _dump_tail():
        pltpu.sync_copy(shacc.at[pl.ds(NS * NPT, NTAIL)],
                        acc_h.at[c, pl.ds(NS * NPT, NTAIL)])

    pltpu.sync_copy(cnt_v, cnt_h.at[pl.ds(pl.multiple_of(w * N, 8), N)])


def _mmT(x, w):
    return lax.dot_general(x, w, (((1,), (1,)), ((), ())),
                           preferred_element_type=jnp.float32)


def _lrelu(v):
    return jnp.where(v > 0, v, 0.01 * v)


def _elu(v):
    return jnp.where(v > 0, v, jnp.exp(jnp.minimum(v, 0.0)) - 1.0)


def _gru_step(h_in, h_state, wih, whh, bih, bhh):
    gi = _mmT(h_in, wih) + bih
    gh = _mmT(h_state, whh) + bhh
    r = jax.nn.sigmoid(gi[:, :H] + gh[:, :H])
    z = jax.nn.sigmoid(gi[:, H:2 * H] + gh[:, H:2 * H])
    n = jnp.tanh(gi[:, 2 * H:] + r * gh[:, 2 * H:])
    return (1.0 - z) * n + z * h_state


def _full(shape):
    return pl.BlockSpec(shape, lambda *_: (0,) * len(shape))


def _tc_pre(raw, l1W, l1b, gnWa, gnb, ga1, gab):
    def body(raw_r, l1W_r, l1b_r, gnWa_r, gnb_r, ga1_r, gab_r,
             x_r, u_r, s1_r):
        x = _lrelu(_mmT(raw_r[...], l1W_r[...]) + l1b_r[...])
        x_r[...] = x
        u_r[...] = _mmT(raw_r[...], gnWa_r[...]) + gnb_r[...]
        s1_r[...] = jnp.sum(x * ga1_r[...], axis=1, keepdims=True) + gab_r[0, 0]

    nblk = N // NB
    return pl.pallas_call(
        body,
        grid=(nblk,),
        in_specs=[
            pl.BlockSpec((NB, D), lambda i: (i, 0)),
            _full((H, D)), _full((1, H)), _full((H, D)), _full((1, H)),
            _full((1, H)), _full((1, 1)),
        ],
        out_specs=[
            pl.BlockSpec((NB, H), lambda i: (i, 0)),
            pl.BlockSpec((NB, H), lambda i: (i, 0)),
            pl.BlockSpec((NB, 1), lambda i: (i, 0)),
        ],
        out_shape=[
            jax.ShapeDtypeStruct((N, H), jnp.float32),
            jax.ShapeDtypeStruct((N, H), jnp.float32),
            jax.ShapeDtypeStruct((N, 1), jnp.float32),
        ],
    )(raw, l1W, l1b, gnWa, gnb, ga1, gab)


def _tc_edge_proj(edge_attr, gnWb):
    def body(ea_r, w_r, v_r):
        v_r[...] = _mmT(ea_r[...], w_r[...])

    return pl.pallas_call(
        body,
        grid=(E // EB,),
        in_specs=[pl.BlockSpec((EB, ED), lambda i: (i, 0)), _full((H, ED))],
        out_specs=pl.BlockSpec((EB, H), lambda i: (i, 0)),
        out_shape=jax.ShapeDtypeStruct((E, H), jnp.float32),
    )(edge_attr, gnWb)


def _tc_xj(us, v, ga2):
    def body(us_r, v_r, ga2_r, xj_r, t_r):
        xj = _lrelu(us_r[...] + v_r[...])
        xj_r[...] = xj
        t_r[...] = jnp.sum(xj * ga2_r[...], axis=1, keepdims=True)

    return pl.pallas_call(
        body,
        grid=(E // EB,),
        in_specs=[
            pl.BlockSpec((EB, H), lambda i: (i, 0)),
            pl.BlockSpec((EB, H), lambda i: (i, 0)),
            _full((1, H)),
        ],
        out_specs=[
            pl.BlockSpec((EB, H), lambda i: (i, 0)),
            pl.BlockSpec((EB, 1), lambda i: (i, 0)),
        ],
        out_shape=[
            jax.ShapeDtypeStruct((E, H), jnp.float32),
            jax.ShapeDtypeStruct((E, 1), jnp.float32),
        ],
    )(us, v, ga2)


def _tc_escore(a, b):
    def body(a_r, b_r, e_r):
        sc = _lrelu(a_r[...] + b_r[...])
        e_r[...] = jnp.exp(sc - jnp.max(sc))

    r = E // 128
    return pl.pallas_call(
        body,
        in_specs=[_full((r, 128)), _full((r, 128))],
        out_specs=_full((r, 128)),
        out_shape=jax.ShapeDtypeStruct((r, 128), jnp.float32),
    )(a, b)


def _tc_msg(e, xs):
    def body(e_r, xs_r, m_r):
        m_r[...] = e_r[...] * xs_r[...]

    return pl.pallas_call(
        body,
        grid=(E // EB,),
        in_specs=[
            pl.BlockSpec((EB, 1), lambda i: (i, 0)),
            pl.BlockSpec((EB, H), lambda i: (i, 0)),
        ],
        out_specs=pl.BlockSpec((EB, H), lambda i: (i, 0)),
        out_shape=jax.ShapeDtypeStruct((E, H), jnp.float32),
    )(e, xs)


def _tc_red_cnt(cnt):
    def body(cnt_r, s_r):
        ones = jnp.ones((NW, 1), jnp.float32)
        s_r[...] = lax.dot_general(cnt_r[...], ones, (((0,), (0,)), ((), ())),
                                   preferred_element_type=jnp.float32)

    return pl.pallas_call(
        body,
        in_specs=[_full((NW, N))],
        out_specs=_full((N, 1)),
        out_shape=jax.ShapeDtypeStruct((N, 1), jnp.float32),
    )(cnt)


def _tc_post(acc, ssum, x, tW, tb, wih, whh, bih, bhh, a1n, a2n, abn):
    def body(acc_r, ssum_r, x_r, tW_r, tb_r, wih_r, whh_r, bih_r, bhh_r,
             a1_r, a2_r, ab_r, xn_r, s1_r, s2_r):
        ssum = ssum_r[...]
        r = 1.0 / (ssum + 1e-16)
        accs = (acc_r[0] + acc_r[1]) * r
        cnt = ssum * r
        h = _elu(_mmT(accs, tW_r[...]) + cnt * tb_r[...])
        xn = jax.nn.relu(_gru_step(h, x_r[...], wih_r[...], whh_r[...],
                                   bih_r[...], bhh_r[...]))
        xn_r[...] = xn
        s1_r[...] = jnp.sum(xn * a1_r[...], axis=1, keepdims=True) + ab_r[0, 0]
        s2_r[...] = jnp.sum(xn * a2_r[...], axis=1, keepdims=True)

    nblk = N // NB
    return pl.pallas_call(
        body,
        grid=(nblk,),
        in_specs=[
            pl.BlockSpec((NC, NB, H), lambda i: (0, i, 0)),
            pl.BlockSpec((NB, 1), lambda i: (i, 0)),
            pl.BlockSpec((NB, H), lambda i: (i, 0)),
            _full((H, H)), _full((1, H)),
            _full((3 * H, H)), _full((3 * H, H)),
            _full((1, 3 * H)), _full((1, 3 * H)),
            _full((1, H)), _full((1, H)), _full((1, 1)),
        ],
        out_specs=[
            pl.BlockSpec((NB, H), lambda i: (i, 0)),
            pl.BlockSpec((NB, 1), lambda i: (i, 0)),
            pl.BlockSpec((NB, 1), lambda i: (i, 0)),
        ],
        out_shape=[
            jax.ShapeDtypeStruct((N, H), jnp.float32),
            jax.ShapeDtypeStruct((N, 1), jnp.float32),
            jax.ShapeDtypeStruct((N, 1), jnp.float32),
        ],
    )(acc, ssum, x, tW, tb, wih, whh, bih, bhh, a1n, a2n, abn)


def _tc_mol(xb, xa, batch2, ma1, ma2, mab, mtW, mtb,
            mwih, mwhh, mbih, mbhh, l2W, l2b):
    NBLK = N // NB
    neg = -1e30

    def body(xb_r, xa_r, b_r, ma1_r, ma2_r, mab_r, mtW_r, mtb_r,
             mwih_r, mwhh_r, mbih_r, mbhh_r, l2W_r, l2b_r, out_r, sc_r):
        iota_g = lax.broadcasted_iota(jnp.int32, (NB, G), 1)

        def maskf(b):
            bb = b_r[pl.ds(b * NB, NB), :]
            return (bb == iota_g).astype(jnp.float32)

        pool = jnp.zeros((G, H), jnp.float32)
        for b in range(NBLK):
            mf = maskf(b)
            pool = pool + lax.dot_general(
                mf, xb_r[pl.ds(b * NB, NB), :], (((0,), (0,)), ((), ())),
                preferred_element_type=jnp.float32)
        out = jax.nn.relu(pool)

        for t in range(2):
            def xm(b):
                xbb = xb_r[pl.ds(b * NB, NB), :]
                if t == 0:
                    return (xbb + xa_r[pl.ds(b * NB, NB), :]) * 0.5
                return xbb

            s1row = _mmT(ma1_r[...], out) + mab_r[0, 0]
            m = jnp.full((1, G), neg, jnp.float32)
            for b in range(NBLK):
                mf = maskf(b)
                s2m = jnp.sum(xm(b) * ma2_r[...], axis=1,
                              keepdims=True)
                g1 = jnp.sum(mf * s1row, axis=1, keepdims=True)
                sc = _lrelu(g1 + s2m)
                sc_r[pl.ds(b * NB, NB), :] = sc
                mw = jnp.where(mf > 0, sc, neg)
                m = jnp.maximum(m, jnp.max(mw, axis=0, keepdims=True))
            m = jnp.where(m > neg * 0.5, m, 0.0)
            ssum = jnp.zeros((1, G), jnp.float32)
            for b in range(NBLK):
                mf = maskf(b)
                mg = jnp.sum(mf * m, axis=1, keepdims=True)
                e = jnp.exp(sc_r[pl.ds(b * NB, NB), :] - mg)
                sc_r[pl.ds(b * NB, NB), :] = e
                ssum = ssum + lax.dot_general(
                    e, mf, (((0,), (0,)), ((), ())),
                    preferred_element_type=jnp.float32)
            accm = jnp.zeros((G, H), jnp.float32)
            cntc = jnp.zeros((G, 1), jnp.float32)
            for b in range(NBLK):
                mf = maskf(b)
                denom = jnp.sum(mf * ssum, axis=1, keepdims=True)
                alpha = sc_r[pl.ds(b * NB, NB), :] / (denom + 1e-16)
                accm = accm + lax.dot_general(
                    mf, alpha * xm(b), (((0,), (0,)), ((), ())),
                    preferred_element_type=jnp.float32)
                cntc = cntc + lax.dot_general(
                    mf, alpha, (((0,), (0,)), ((), ())),
                    preferred_element_type=jnp.float32)
            h = _elu(_mmT(accm, mtW_r[...]) + cntc * mtb_r[...])
            out = jax.nn.relu(_gru_step(h, out, mwih_r[...], mwhh_r[...],
                                        mbih_r[...], mbhh_r[...]))

        out_r[...] = _mmT(out, l2W_r[...]) + l2b_r[...]

    return pl.pallas_call(
        body,
        in_specs=[
            _full((N, H)), _full((N, H)), _full((N, 1)),
            _full((1, H)), _full((1, H)), _full((1, 1)),
            _full((H, H)), _full((1, H)),
            _full((3 * H, H)), _full((3 * H, H)),
            _full((1, 3 * H)), _full((1, 3 * H)),
            _full((OUT, H)), _full((1, OUT)),
        ],
        out_specs=_full((G, OUT)),
        out_shape=jax.ShapeDtypeStruct((G, OUT), jnp.float32),
        scratch_shapes=[pltpu.VMEM((N, 1), jnp.float32)],
    )(xb, xa, batch2, ma1, ma2, mab, mtW, mtb,
      mwih, mwhh, mbih, mbhh, l2W, l2b)


def _row(v):
    return v.reshape(1, -1)


def kernel(raw, edge_index, edge_attr, batch, params):
    p = params
    src = edge_index[0]
    dst = edge_index[1]
    zrows = jnp.zeros((NPT, H), jnp.float32)
    r2 = E // 128

    ga1 = _row(p['gate_align_W'][0, :H])
    ga2 = _row(p['gate_align_W'][0, H:])
    gab = p['gate_align_b'].reshape(1, 1)
    x0, u, s1g = _tc_pre(raw, p['lin1_W'], _row(p['lin1_b']),
                         p['gate_nei_W'][:, :D], _row(p['gate_nei_b']),
                         ga1, gab)
    v = _tc_edge_proj(edge_attr, p['gate_nei_W'][:, D:])
    us = _sc_gather_rows(u, src)
    xj, t = _tc_xj(us, v, ga2)

    s1d = _sc_gather1(s1g.reshape(N), dst)
    e = _tc_escore(s1d.reshape(r2, 128), t.reshape(r2, 128))
    ef = e.reshape(E)
    msg = _tc_msg(ef.reshape(E, 1), xj)
    acc, cnt = _sc_scatter_rows(msg, ef, dst, zrows)
    ssum = _tc_red_cnt(cnt.reshape(NW, N))
    c1a1 = _row(p['conv1_align_W'][0, :H])
    c1a2 = _row(p['conv1_align_W'][0, H:])
    x1, s1, s2 = _tc_post(acc, ssum, x0,
                          p['gate_attend_W'], _row(p['gate_attend_b']),
                          p['gru0_wih'], p['gru0_whh'],
                          _row(p['gru0_bih']), _row(p['gru0_bhh']),
                          c1a1, c1a2, p['conv1_align_b'].reshape(1, 1))

    xs_in = x1
    for i in (1, 2):
        s1d, s2s = _sc_gather2(s1.reshape(N), dst, s2.reshape(N), src)
        e = _tc_escore(s1d.reshape(r2, 128), s2s.reshape(r2, 128))
        ef = e.reshape(E)
        xs = _sc_gather_rows(xs_in, src)
        msg = _tc_msg(ef.reshape(E, 1), xs)
        acc, cnt = _sc_scatter_rows(msg, ef, dst, zrows)
        ssum = _tc_red_cnt(cnt.reshape(NW, N))
        if i == 1:
            na1 = _row(p['conv2_align_W'][0, :H])
            na2 = _row(p['conv2_align_W'][0, H:])
            nab = p['conv2_align_b'].reshape(1, 1)
        else:
            na1, na2, nab = c1a1, c1a2, p['conv1_align_b'].reshape(1, 1)
        xs_in, s1, s2 = _tc_post(
            acc, ssum, xs_in,
            p['conv%d_attend_W' % i], _row(p['conv%d_attend_b' % i]),
            p['gru%d_wih' % i], p['gru%d_whh' % i],
            _row(p['gru%d_bih' % i]), _row(p['gru%d_bhh' % i]),
            na1, na2, nab)
        if i == 1:
            xa = xs_in
    xb = xs_in

    return _tc_mol(xb, xa, batch.reshape(N, 1),
                   _row(p['mol_align_W'][0, :H]), _row(p['mol_align_W'][0, H:]),
                   p['mol_align_b'].reshape(1, 1),
                   p['mol_attend_W'], _row(p['mol_attend_b']),
                   p['mgru_wih'], p['mgru_whh'],
                   _row(p['mgru_bih']), _row(p['mgru_bhh']),
                   p['lin2_W'], _row(p['lin2_b']))


# speedup vs baseline: 11.7016x; 1.4583x over previous
import functools

import jax
import jax.numpy as jnp
from jax import lax
from jax.experimental import pallas as pl
from jax.experimental.pallas import tpu as pltpu
from jax.experimental.pallas import tpu_sc as plsc

N = 10000
E = 320000
D = 128
ED = 16
G = 256
H = 128
OUT = 128

NC = 2
NS = 16
NW = NC * NS
EPW = E // NW
LN = 16
CH = 80
NCHK = EPW // CH
NPT = 624
NTAIL = N - NS * NPT

EB = 2000
NB = 2000

def _wid():
    return lax.axis_index("s") * NC + lax.axis_index("c")


@functools.cache
def _mesh():
    return plsc.VectorSubcoreMesh(
        core_axis_name="c", subcore_axis_name="s",
        num_cores=NC, num_subcores=NS)


def _sc_lrelu(v):
    return jnp.where(v > 0, v, 0.01 * v)


@functools.cache
def _build_escore_conv():
  return functools.partial(
    pl.kernel,
    out_type=(jax.ShapeDtypeStruct((E,), jnp.float32),
              jax.ShapeDtypeStruct((NW * N,), jnp.float32)),
    mesh=_mesh(),
    compiler_params=pltpu.CompilerParams(needs_layout_passes=False),
    scratch_types=[
        pltpu.VMEM((N,), jnp.float32),
        pltpu.VMEM((N,), jnp.float32),
        pltpu.VMEM((EPW,), jnp.int32),
        pltpu.VMEM((EPW,), jnp.int32),
        pltpu.VMEM((LN,), jnp.float32),
        pltpu.VMEM((LN,), jnp.float32),
        pltpu.VMEM((EPW,), jnp.float32),
        pltpu.VMEM((N,), jnp.float32),
    ],
  )(_escore_conv_body)


def _sc_escore_conv(s1, s2, srcf, dstf, mx1, mx2):
    return _build_escore_conv()(s1, s2, srcf, dstf, mx1, mx2)


def _escore_conv_body(s1_h, s2_h, src_h, dst_h, mx1_h, mx2_h, e_h, cnt_h,
                      s1_v, s2_v, src_v, dst_v, mx1_v, mx2_v, e_v, cnt_v):
    w = _wid()
    base = w * EPW
    pltpu.sync_copy(s1_h, s1_v)
    pltpu.sync_copy(s2_h, s2_v)
    pltpu.sync_copy(src_h.at[pl.ds(base, EPW)], src_v)
    pltpu.sync_copy(dst_h.at[pl.ds(base, EPW)], dst_v)
    pltpu.sync_copy(mx1_h, mx1_v)
    pltpu.sync_copy(mx2_h, mx2_v)

    def zero(i, carry):
        cnt_v[pl.ds(i * LN, LN)] = jnp.zeros((LN,), jnp.float32)
        return carry

    lax.fori_loop(0, N // LN, zero, 0)
    m = _sc_lrelu(mx1_v[...] + mx2_v[...])

    def body(i, carry):
        sl = pl.ds(i * LN, LN)
        s1d = plsc.load_gather(s1_v, [dst_v[sl]])
        s2s = plsc.load_gather(s2_v, [src_v[sl]])
        ev = jnp.exp(_sc_lrelu(s1d + s2s) - m)
        e_v[sl] = ev
        plsc.addupdate_scatter(cnt_v, [dst_v[sl]], ev)
        return carry

    lax.fori_loop(0, EPW // LN, body, 0)
    pltpu.sync_copy(e_v, e_h.at[pl.ds(base, EPW)])
    pltpu.sync_copy(cnt_v, cnt_h.at[pl.ds(pl.multiple_of(w * N, 8), N)])


@functools.cache
def _build_escore_gate():
  return functools.partial(
    pl.kernel,
    out_type=(jax.ShapeDtypeStruct((E,), jnp.float32),
              jax.ShapeDtypeStruct((NW * N,), jnp.float32)),
    mesh=_mesh(),
    compiler_params=pltpu.CompilerParams(needs_layout_passes=False),
    scratch_types=[
        pltpu.VMEM((N,), jnp.float32),
        pltpu.VMEM((EPW,), jnp.float32),
        pltpu.VMEM((EPW,), jnp.int32),
        pltpu.VMEM((LN,), jnp.float32),
        pltpu.VMEM((LN,), jnp.float32),
        pltpu.VMEM((EPW,), jnp.float32),
        pltpu.VMEM((N,), jnp.float32),
    ],
  )(_escore_gate_body)


def _sc_escore_gate(s1, t, dstf, mx1, mx2):
    return _build_escore_gate()(s1, t, dstf, mx1, mx2)


def _escore_gate_body(s1_h, t_h, dst_h, mx1_h, mx2_h, e_h, cnt_h,
                      s1_v, t_v, dst_v, mx1_v, mx2_v, e_v, cnt_v):
    w = _wid()
    base = w * EPW
    pltpu.sync_copy(s1_h, s1_v)
    pltpu.sync_copy(t_h.at[pl.ds(base, EPW)], t_v)
    pltpu.sync_copy(dst_h.at[pl.ds(base, EPW)], dst_v)
    pltpu.sync_copy(mx1_h, mx1_v)
    pltpu.sync_copy(mx2_h, mx2_v)

    def zero(i, carry):
        cnt_v[pl.ds(i * LN, LN)] = jnp.zeros((LN,), jnp.float32)
        return carry

    lax.fori_loop(0, N // LN, zero, 0)
    m = _sc_lrelu(mx1_v[...] + mx2_v[...])

    def body(i, carry):
        sl = pl.ds(i * LN, LN)
        s1d = plsc.load_gather(s1_v, [dst_v[sl]])
        ev = jnp.exp(_sc_lrelu(s1d + t_v[sl]) - m)
        e_v[sl] = ev
        plsc.addupdate_scatter(cnt_v, [dst_v[sl]], ev)
        return carry

    lax.fori_loop(0, EPW // LN, body, 0)
    pltpu.sync_copy(e_v, e_h.at[pl.ds(base, EPW)])
    pltpu.sync_copy(cnt_v, cnt_h.at[pl.ds(pl.multiple_of(w * N, 8), N)])


@functools.cache
def _build_gather_rows():
  return functools.partial(
    pl.kernel,
    out_type=jax.ShapeDtypeStruct((E, H), jnp.float32),
    mesh=_mesh(),
    compiler_params=pltpu.CompilerParams(needs_layout_passes=False),
    scratch_types=[
        pltpu.VMEM((EPW,), jnp.int32),
        pltpu.VMEM((CH, H), jnp.float32),
        pltpu.SemaphoreType.DMA,
    ],
  )(_gather_rows_body)


def _sc_gather_rows(x, idx):
    return _build_gather_rows()(x, idx)


def _gather_rows_body(x_h, idx_h, out_h, idx_v, rows_v, sem):
    base = _wid() * EPW
    pltpu.sync_copy(idx_h.at[pl.ds(base, EPW)], idx_v)

    def chunk(k, carry):
        cp = pltpu.async_copy(x_h.at[idx_v.at[pl.ds(k * CH, CH)]], rows_v, sem)
        cp.wait()
        pltpu.sync_copy(rows_v, out_h.at[pl.ds(base + k * CH, CH)])
        return carry

    lax.fori_loop(0, NCHK, chunk, 0)


def _agg_prologue(z_h, shacc):
    s = lax.axis_index("s")
    off = pl.multiple_of(s * NPT, 8)
    pltpu.sync_copy(z_h.at[pl.ds(0, NPT)], shacc.at[pl.ds(off, NPT)])

    @pl.when(s == 0)
    def _zero_tail():
        pltpu.sync_copy(z_h.at[pl.ds(0, NTAIL)],
                        shacc.at[pl.ds(NS * NPT, NTAIL)])

    plsc.subcore_barrier()
    return off


def _agg_epilogue(off, acc_h, shacc):
    c = lax.axis_index("c")
    s = lax.axis_index("s")
    plsc.subcore_barrier()
    pltpu.sync_copy(shacc.at[pl.ds(off, NPT)],
                    acc_h.at[c, pl.ds(off, NPT)])

    @pl.when(s == 0)
    def _dump_tail():
        pltpu.sync_copy(shacc.at[pl.ds(NS * NPT, NTAIL)],
                        acc_h.at[c, pl.ds(NS * NPT, NTAIL)])


def _scale_rows(rows_v, ec_v):
    def srow(r, cc):
        ev = plsc.load_gather(ec_v, [jnp.full((LN,), r, jnp.int32)])
        for j in range(H // LN):
            sl = pl.ds(j * LN, LN)
            rows_v[r, sl] = rows_v[r, sl] * ev
        return cc

    lax.fori_loop(0, CH, srow, 0)


@functools.cache
def _build_agg_gather():
  return functools.partial(
    pl.kernel,
    out_type=jax.ShapeDtypeStruct((NC, N, H), jnp.float32),
    mesh=_mesh(),
    compiler_params=pltpu.CompilerParams(needs_layout_passes=False),
    scratch_types=[
        pltpu.VMEM((EPW,), jnp.int32),
        pltpu.VMEM((EPW,), jnp.int32),
        pltpu.VMEM((CH,), jnp.float32),
        pltpu.VMEM((CH, H), jnp.float32),
        pltpu.SemaphoreType.DMA,
        pltpu.VMEM_SHARED((N, H), jnp.float32),
    ],
  )(_agg_gather_body)


def _sc_agg_gather(x, e, srcf, dstf, z):
    return _build_agg_gather()(x, e, srcf, dstf, z)


def _agg_gather_body(x_h, e_h, src_h, dst_h, z_h, acc_h,
                     src_v, dst_v, ec_v, rows_v, sem, shacc):
    w = _wid()
    base = w * EPW
    pltpu.sync_copy(src_h.at[pl.ds(base, EPW)], src_v)
    pltpu.sync_copy(dst_h.at[pl.ds(base, EPW)], dst_v)
    off = _agg_prologue(z_h, shacc)

    def chunk(k, carry):
        cp = pltpu.async_copy(x_h.at[src_v.at[pl.ds(k * CH, CH)]], rows_v, sem)
        pltpu.sync_copy(e_h.at[pl.ds(base + k * CH, CH)], ec_v)
        cp.wait()
        _scale_rows(rows_v, ec_v)
        pltpu.sync_copy(rows_v, shacc.at[dst_v.at[pl.ds(k * CH, CH)]],
                        add=True)
        return carry

    lax.fori_loop(0, NCHK, chunk, 0)
    _agg_epilogue(off, acc_h, shacc)


@functools.cache
def _build_agg_stream():
  return functools.partial(
    pl.kernel,
    out_type=jax.ShapeDtypeStruct((NC, N, H), jnp.float32),
    mesh=_mesh(),
    compiler_params=pltpu.CompilerParams(needs_layout_passes=False),
    scratch_types=[
        pltpu.VMEM((EPW,), jnp.int32),
        pltpu.VMEM((CH,), jnp.float32),
        pltpu.VMEM((CH, H), jnp.float32),
        pltpu.SemaphoreType.DMA,
        pltpu.VMEM_SHARED((N, H), jnp.float32),
    ],
  )(_agg_stream_body)


def _sc_agg_stream(msg, e, dstf, z):
    return _build_agg_stream()(msg, e, dstf, z)


def _agg_stream_body(msg_h, e_h, dst_h, z_h, acc_h,
                     dst_v, ec_v, rows_v, sem, shacc):
    w = _wid()
    base = w * EPW
    pltpu.sync_copy(dst_h.at[pl.ds(base, EPW)], dst_v)
    off = _agg_prologue(z_h, shacc)

    def chunk(k, carry):
        cp = pltpu.async_copy(msg_h.at[pl.ds(base + k * CH, CH)], rows_v, sem)
        pltpu.sync_copy(e_h.at[pl.ds(base + k * CH, CH)], ec_v)
        cp.wait()
        _scale_rows(rows_v, ec_v)
        pltpu.sync_copy(rows_v, shacc.at[dst_v.at[pl.ds(k * CH, CH)]],
                        add=True)
        return carry

    lax.fori_loop(0, NCHK, chunk, 0)
    _agg_epilogue(off, acc_h, shacc)


def _mmT(x, w):
    return lax.dot_general(x, w, (((1,), (1,)), ((), ())),
                           preferred_element_type=jnp.float32)


def _lrelu(v):
    return jnp.where(v > 0, v, 0.01 * v)


def _elu(v):
    return jnp.where(v > 0, v, jnp.exp(jnp.minimum(v, 0.0)) - 1.0)


def _gru_step(h_in, h_state, wih, whh, bih, bhh):
    gi = _mmT(h_in, wih) + bih
    gh = _mmT(h_state, whh) + bhh
    r = jax.nn.sigmoid(gi[:, :H] + gh[:, :H])
    z = jax.nn.sigmoid(gi[:, H:2 * H] + gh[:, H:2 * H])
    n = jnp.tanh(gi[:, 2 * H:] + r * gh[:, 2 * H:])
    return (1.0 - z) * n + z * h_state


def _full(shape):
    return pl.BlockSpec(shape, lambda *_: (0,) * len(shape))


def _tc_pre(raw, l1W, l1b, gnWa, gnb, ga1, gab):
    def body(raw_r, l1W_r, l1b_r, gnWa_r, gnb_r, ga1_r, gab_r,
             x_r, u_r, s1_r, mx_r):
        x = _lrelu(_mmT(raw_r[...], l1W_r[...]) + l1b_r[...])
        x_r[...] = x
        u_r[...] = _mmT(raw_r[...], gnWa_r[...]) + gnb_r[...]
        s1 = jnp.sum(x * ga1_r[...], axis=1, keepdims=True) + gab_r[0, 0]
        s1_r[...] = s1

        @pl.when(pl.program_id(0) == 0)
        def _init():
            mx_r[...] = jnp.full((1, 1), -1e30, jnp.float32)

        mx_r[...] = jnp.maximum(mx_r[...], jnp.max(s1))

    nblk = N // NB
    return pl.pallas_call(
        body,
        grid=(nblk,),
        in_specs=[
            pl.BlockSpec((NB, D), lambda i: (i, 0)),
            _full((H, D)), _full((1, H)), _full((H, D)), _full((1, H)),
            _full((1, H)), _full((1, 1)),
        ],
        out_specs=[
            pl.BlockSpec((NB, H), lambda i: (i, 0)),
            pl.BlockSpec((NB, H), lambda i: (i, 0)),
            pl.BlockSpec((NB, 1), lambda i: (i, 0)),
            pl.BlockSpec((1, 1), lambda i: (0, 0)),
        ],
        out_shape=[
            jax.ShapeDtypeStruct((N, H), jnp.float32),
            jax.ShapeDtypeStruct((N, H), jnp.float32),
            jax.ShapeDtypeStruct((N, 1), jnp.float32),
            jax.ShapeDtypeStruct((1, 1), jnp.float32),
        ],
    )(raw, l1W, l1b, gnWa, gnb, ga1, gab)


def _tc_edge_proj(edge_attr, gnWb):
    def body(ea_r, w_r, v_r):
        v_r[...] = _mmT(ea_r[...], w_r[...])

    return pl.pallas_call(
        body,
        grid=(E // EB,),
        in_specs=[pl.BlockSpec((EB, ED), lambda i: (i, 0)), _full((H, ED))],
        out_specs=pl.BlockSpec((EB, H), lambda i: (i, 0)),
        out_shape=jax.ShapeDtypeStruct((E, H), jnp.float32),
    )(edge_attr, gnWb)


def _tc_xj(us, v, ga2):
    def body(us_r, v_r, ga2_r, xj_r, t_r, mx_r):
        xj = _lrelu(us_r[...] + v_r[...])
        xj_r[...] = xj
        t = jnp.sum(xj * ga2_r[...], axis=1, keepdims=True)
        t_r[...] = t

        @pl.when(pl.program_id(0) == 0)
        def _init():
            mx_r[...] = jnp.full((1, 1), -1e30, jnp.float32)

        mx_r[...] = jnp.maximum(mx_r[...], jnp.max(t))

    return pl.pallas_call(
        body,
        grid=(E // EB,),
        in_specs=[
            pl.BlockSpec((EB, H), lambda i: (i, 0)),
            pl.BlockSpec((EB, H), lambda i: (i, 0)),
            _full((1, H)),
        ],
        out_specs=[
            pl.BlockSpec((EB, H), lambda i: (i, 0)),
            pl.BlockSpec((EB, 1), lambda i: (i, 0)),
            pl.BlockSpec((1, 1), lambda i: (0, 0)),
        ],
        out_shape=[
            jax.ShapeDtypeStruct((E, H), jnp.float32),
            jax.ShapeDtypeStruct((E, 1), jnp.float32),
            jax.ShapeDtypeStruct((1, 1), jnp.float32),
        ],
    )(us, v, ga2)


def _tc_red_cnt(cnt):
    def body(cnt_r, s_r):
        ones = jnp.ones((NW, 1), jnp.float32)
        s_r[...] = lax.dot_general(cnt_r[...], ones, (((0,), (0,)), ((), ())),
                                   preferred_element_type=jnp.float32)

    return pl.pallas_call(
        body,
        in_specs=[_full((NW, N))],
        out_specs=_full((N, 1)),
        out_shape=jax.ShapeDtypeStruct((N, 1), jnp.float32),
    )(cnt)


def _tc_post(acc, ssum, x, tW, tb, wih, whh, bih, bhh, a1n, a2n, abn):
    def body(acc_r, ssum_r, x_r, tW_r, tb_r, wih_r, whh_r, bih_r, bhh_r,
             a1_r, a2_r, ab_r, xn_r, s1_r, s2_r, mx1_r, mx2_r):
        ssum = ssum_r[...]
        r = 1.0 / (ssum + 1e-16)
        accs = (acc_r[0] + acc_r[1]) * r
        cnt = ssum * r
        h = _elu(_mmT(accs, tW_r[...]) + cnt * tb_r[...])
        xn = jax.nn.relu(_gru_step(h, x_r[...], wih_r[...], whh_r[...],
                                   bih_r[...], bhh_r[...]))
        xn_r[...] = xn
        s1 = jnp.sum(xn * a1_r[...], axis=1, keepdims=True) + ab_r[0, 0]
        s2 = jnp.sum(xn * a2_r[...], axis=1, keepdims=True)
        s1_r[...] = s1
        s2_r[...] = s2

        @pl.when(pl.program_id(0) == 0)
        def _init():
            mx1_r[...] = jnp.full((1, 1), -1e30, jnp.float32)
            mx2_r[...] = jnp.full((1, 1), -1e30, jnp.float32)

        mx1_r[...] = jnp.maximum(mx1_r[...], jnp.max(s1))
        mx2_r[...] = jnp.maximum(mx2_r[...], jnp.max(s2))

    nblk = N // NB
    return pl.pallas_call(
        body,
        grid=(nblk,),
        in_specs=[
            pl.BlockSpec((NC, NB, H), lambda i: (0, i, 0)),
            pl.BlockSpec((NB, 1), lambda i: (i, 0)),
            pl.BlockSpec((NB, H), lambda i: (i, 0)),
            _full((H, H)), _full((1, H)),
            _full((3 * H, H)), _full((3 * H, H)),
            _full((1, 3 * H)), _full((1, 3 * H)),
            _full((1, H)), _full((1, H)), _full((1, 1)),
        ],
        out_specs=[
            pl.BlockSpec((NB, H), lambda i: (i, 0)),
            pl.BlockSpec((NB, 1), lambda i: (i, 0)),
            pl.BlockSpec((NB, 1), lambda i: (i, 0)),
            pl.BlockSpec((1, 1), lambda i: (0, 0)),
            pl.BlockSpec((1, 1), lambda i: (0, 0)),
        ],
        out_shape=[
            jax.ShapeDtypeStruct((N, H), jnp.float32),
            jax.ShapeDtypeStruct((N, 1), jnp.float32),
            jax.ShapeDtypeStruct((N, 1), jnp.float32),
            jax.ShapeDtypeStruct((1, 1), jnp.float32),
            jax.ShapeDtypeStruct((1, 1), jnp.float32),
        ],
    )(acc, ssum, x, tW, tb, wih, whh, bih, bhh, a1n, a2n, abn)


def _tc_mol(xb, xa, batch2, ma1, ma2, mab, mtW, mtb,
            mwih, mwhh, mbih, mbhh, l2W, l2b):
    NBLK = N // NB
    neg = -1e30

    def body(xb_r, xa_r, b_r, ma1_r, ma2_r, mab_r, mtW_r, mtb_r,
             mwih_r, mwhh_r, mbih_r, mbhh_r, l2W_r, l2b_r, out_r, sc_r):
        iota_g = lax.broadcasted_iota(jnp.int32, (NB, G), 1)

        def maskf(b):
            bb = b_r[pl.ds(b * NB, NB), :]
            return (bb == iota_g).astype(jnp.float32)

        pool = jnp.zeros((G, H), jnp.float32)
        for b in range(NBLK):
            mf = maskf(b)
            pool = pool + lax.dot_general(
                mf, xb_r[pl.ds(b * NB, NB), :], (((0,), (0,)), ((), ())),
                preferred_element_type=jnp.float32)
        out = jax.nn.relu(pool)

        for t in range(2):
            def xm(b):
                xbb = xb_r[pl.ds(b * NB, NB), :]
                if t == 0:
                    return (xbb + xa_r[pl.ds(b * NB, NB), :]) * 0.5
                return xbb

            s1row = _mmT(ma1_r[...], out) + mab_r[0, 0]
            m = jnp.full((1, G), neg, jnp.float32)
            for b in range(NBLK):
                mf = maskf(b)
                s2m = jnp.sum(xm(b) * ma2_r[...], axis=1,
                              keepdims=True)
                g1 = jnp.sum(mf * s1row, axis=1, keepdims=True)
                sc = _lrelu(g1 + s2m)
                sc_r[pl.ds(b * NB, NB), :] = sc
                mw = jnp.where(mf > 0, sc, neg)
                m = jnp.maximum(m, jnp.max(mw, axis=0, keepdims=True))
            m = jnp.where(m > neg * 0.5, m, 0.0)
            ssum = jnp.zeros((1, G), jnp.float32)
            for b in range(NBLK):
                mf = maskf(b)
                mg = jnp.sum(mf * m, axis=1, keepdims=True)
                e = jnp.exp(sc_r[pl.ds(b * NB, NB), :] - mg)
                sc_r[pl.ds(b * NB, NB), :] = e
                ssum = ssum + lax.dot_general(
                    e, mf, (((0,), (0,)), ((), ())),
                    preferred_element_type=jnp.float32)
            accm = jnp.zeros((G, H), jnp.float32)
            cntc = jnp.zeros((G, 1), jnp.float32)
            for b in range(NBLK):
                mf = maskf(b)
                denom = jnp.sum(mf * ssum, axis=1, keepdims=True)
                alpha = sc_r[pl.ds(b * NB, NB), :] / (denom + 1e-16)
                accm = accm + lax.dot_general(
                    mf, alpha * xm(b), (((0,), (0,)), ((), ())),
                    preferred_element_type=jnp.float32)
                cntc = cntc + lax.dot_general(
                    mf, alpha, (((0,), (0,)), ((), ())),
                    preferred_element_type=jnp.float32)
            h = _elu(_mmT(accm, mtW_r[...]) + cntc * mtb_r[...])
            out = jax.nn.relu(_gru_step(h, out, mwih_r[...], mwhh_r[...],
                                        mbih_r[...], mbhh_r[...]))

        out_r[...] = _mmT(out, l2W_r[...]) + l2b_r[...]

    return pl.pallas_call(
        body,
        in_specs=[
            _full((N, H)), _full((N, H)), _full((N, 1)),
            _full((1, H)), _full((1, H)), _full((1, 1)),
            _full((H, H)), _full((1, H)),
            _full((3 * H, H)), _full((3 * H, H)),
            _full((1, 3 * H)), _full((1, 3 * H)),
            _full((OUT, H)), _full((1, OUT)),
        ],
        out_specs=_full((G, OUT)),
        out_shape=jax.ShapeDtypeStruct((G, OUT), jnp.float32),
        scratch_shapes=[pltpu.VMEM((N, 1), jnp.float32)],
    )(xb, xa, batch2, ma1, ma2, mab, mtW, mtb,
      mwih, mwhh, mbih, mbhh, l2W, l2b)


def _row(v):
    return v.reshape(1, -1)


def _b16(m):
    return jnp.broadcast_to(m.reshape(1), (LN,))


def kernel(raw, edge_index, edge_attr, batch, params):
    p = params
    src = edge_index[0]
    dst = edge_index[1]
    zrows = jnp.zeros((NPT, H), jnp.float32)

    ga1 = _row(p['gate_align_W'][0, :H])
    ga2 = _row(p['gate_align_W'][0, H:])
    gab = p['gate_align_b'].reshape(1, 1)
    x0, u, s1g, mx1g = _tc_pre(raw, p['lin1_W'], _row(p['lin1_b']),
                               p['gate_nei_W'][:, :D], _row(p['gate_nei_b']),
                               ga1, gab)
    v = _tc_edge_proj(edge_attr, p['gate_nei_W'][:, D:])
    us = _sc_gather_rows(u, src)
    xj, t, mxt = _tc_xj(us, v, ga2)

    ef, cnt = _sc_escore_gate(s1g.reshape(N), t.reshape(E), dst,
                              _b16(mx1g), _b16(mxt))
    acc = _sc_agg_stream(xj, ef, dst, zrows)
    ssum = _tc_red_cnt(cnt.reshape(NW, N))
    c1a1 = _row(p['conv1_align_W'][0, :H])
    c1a2 = _row(p['conv1_align_W'][0, H:])
    x1, s1, s2, mx1, mx2 = _tc_post(
        acc, ssum, x0,
        p['gate_attend_W'], _row(p['gate_attend_b']),
        p['gru0_wih'], p['gru0_whh'],
        _row(p['gru0_bih']), _row(p['gru0_bhh']),
        c1a1, c1a2, p['conv1_align_b'].reshape(1, 1))

    xs_in = x1
    for i in (1, 2):
        ef, cnt = _sc_escore_conv(s1.reshape(N), s2.reshape(N), src, dst,
                                  _b16(mx1), _b16(mx2))
        acc = _sc_agg_gather(xs_in, ef, src, dst, zrows)
        ssum = _tc_red_cnt(cnt.reshape(NW, N))
        if i == 1:
            na1 = _row(p['conv2_align_W'][0, :H])
            na2 = _row(p['conv2_align_W'][0, H:])
            nab = p['conv2_align_b'].reshape(1, 1)
        else:
            na1, na2, nab = c1a1, c1a2, p['conv1_align_b'].reshape(1, 1)
        xs_in, s1, s2, mx1, mx2 = _tc_post(
            acc, ssum, xs_in,
            p['conv%d_attend_W' % i], _row(p['conv%d_attend_b' % i]),
            p['gru%d_wih' % i], p['gru%d_whh' % i],
            _row(p['gru%d_bih' % i]), _row(p['gru%d_bhh' % i]),
            na1, na2, nab)
        if i == 1:
            xa = xs_in
    xb = xs_in

    return _tc_mol(xb, xa, batch.reshape(N, 1),
                   _row(p['mol_align_W'][0, :H]), _row(p['mol_align_W'][0, H:]),
                   p['mol_align_b'].reshape(1, 1),
                   p['mol_attend_W'], _row(p['mol_attend_b']),
                   p['mgru_wih'], p['mgru_whh'],
                   _row(p['mgru_bih']), _row(p['mgru_bhh']),
                   p['lin2_W'], _row(p['lin2_b']))


# speedup vs baseline: 14.4908x; 1.2384x over previous
import functools

import jax
import jax.numpy as jnp
from jax import lax
from jax.experimental import pallas as pl
from jax.experimental.pallas import tpu as pltpu
from jax.experimental.pallas import tpu_sc as plsc

N = 10000
E = 320000
D = 128
ED = 16
G = 256
H = 128
OUT = 128

NC = 2
NS = 16
NW = NC * NS
EPW = E // NW
LN = 16
CH = 80
NCHK = EPW // CH
NPT = 624
NTAIL = N - NS * NPT

EB = 2000
NB = 2000

def _wid():
    return lax.axis_index("s") * NC + lax.axis_index("c")


@functools.cache
def _mesh():
    return plsc.VectorSubcoreMesh(
        core_axis_name="c", subcore_axis_name="s",
        num_cores=NC, num_subcores=NS)


def _sc_lrelu(v):
    return jnp.where(v > 0, v, 0.01 * v)


@functools.cache
def _build_escore_conv():
  return functools.partial(
    pl.kernel,
    out_type=(jax.ShapeDtypeStruct((E,), jnp.float32),
              jax.ShapeDtypeStruct((NW * N,), jnp.float32)),
    mesh=_mesh(),
    compiler_params=pltpu.CompilerParams(needs_layout_passes=False),
    scratch_types=[
        pltpu.VMEM((N,), jnp.float32),
        pltpu.VMEM((N,), jnp.float32),
        pltpu.VMEM((EPW,), jnp.int32),
        pltpu.VMEM((EPW,), jnp.int32),
        pltpu.VMEM((LN,), jnp.float32),
        pltpu.VMEM((LN,), jnp.float32),
        pltpu.VMEM((EPW,), jnp.float32),
        pltpu.VMEM((N,), jnp.float32),
    ],
  )(_escore_conv_body)


def _sc_escore_conv(s1, s2, srcf, dstf, mx1, mx2):
    return _build_escore_conv()(s1, s2, srcf, dstf, mx1, mx2)


def _escore_conv_body(s1_h, s2_h, src_h, dst_h, mx1_h, mx2_h, e_h, cnt_h,
                      s1_v, s2_v, src_v, dst_v, mx1_v, mx2_v, e_v, cnt_v):
    w = _wid()
    base = w * EPW
    pltpu.sync_copy(s1_h, s1_v)
    pltpu.sync_copy(s2_h, s2_v)
    pltpu.sync_copy(src_h.at[pl.ds(base, EPW)], src_v)
    pltpu.sync_copy(dst_h.at[pl.ds(base, EPW)], dst_v)
    pltpu.sync_copy(mx1_h, mx1_v)
    pltpu.sync_copy(mx2_h, mx2_v)

    def zero(i, carry):
        cnt_v[pl.ds(i * LN, LN)] = jnp.zeros((LN,), jnp.float32)
        return carry

    lax.fori_loop(0, N // LN, zero, 0)
    m = _sc_lrelu(mx1_v[...] + mx2_v[...])

    def body(i, carry):
        sl = pl.ds(i * LN, LN)
        s1d = plsc.load_gather(s1_v, [dst_v[sl]])
        s2s = plsc.load_gather(s2_v, [src_v[sl]])
        ev = jnp.exp(_sc_lrelu(s1d + s2s) - m)
        e_v[sl] = ev
        plsc.addupdate_scatter(cnt_v, [dst_v[sl]], ev)
        return carry

    lax.fori_loop(0, EPW // LN, body, 0)
    pltpu.sync_copy(e_v, e_h.at[pl.ds(base, EPW)])
    pltpu.sync_copy(cnt_v, cnt_h.at[pl.ds(pl.multiple_of(w * N, 8), N)])


@functools.cache
def _build_escore_gate():
  return functools.partial(
    pl.kernel,
    out_type=(jax.ShapeDtypeStruct((E,), jnp.float32),
              jax.ShapeDtypeStruct((NW * N,), jnp.float32)),
    mesh=_mesh(),
    compiler_params=pltpu.CompilerParams(needs_layout_passes=False),
    scratch_types=[
        pltpu.VMEM((N,), jnp.float32),
        pltpu.VMEM((EPW,), jnp.float32),
        pltpu.VMEM((EPW,), jnp.int32),
        pltpu.VMEM((LN,), jnp.float32),
        pltpu.VMEM((LN,), jnp.float32),
        pltpu.VMEM((EPW,), jnp.float32),
        pltpu.VMEM((N,), jnp.float32),
    ],
  )(_escore_gate_body)


def _sc_escore_gate(s1, t, dstf, mx1, mx2):
    return _build_escore_gate()(s1, t, dstf, mx1, mx2)


def _escore_gate_body(s1_h, t_h, dst_h, mx1_h, mx2_h, e_h, cnt_h,
                      s1_v, t_v, dst_v, mx1_v, mx2_v, e_v, cnt_v):
    w = _wid()
    base = w * EPW
    pltpu.sync_copy(s1_h, s1_v)
    pltpu.sync_copy(t_h.at[pl.ds(base, EPW)], t_v)
    pltpu.sync_copy(dst_h.at[pl.ds(base, EPW)], dst_v)
    pltpu.sync_copy(mx1_h, mx1_v)
    pltpu.sync_copy(mx2_h, mx2_v)

    def zero(i, carry):
        cnt_v[pl.ds(i * LN, LN)] = jnp.zeros((LN,), jnp.float32)
        return carry

    lax.fori_loop(0, N // LN, zero, 0)
    m = _sc_lrelu(mx1_v[...] + mx2_v[...])

    def body(i, carry):
        sl = pl.ds(i * LN, LN)
        s1d = plsc.load_gather(s1_v, [dst_v[sl]])
        ev = jnp.exp(_sc_lrelu(s1d + t_v[sl]) - m)
        e_v[sl] = ev
        plsc.addupdate_scatter(cnt_v, [dst_v[sl]], ev)
        return carry

    lax.fori_loop(0, EPW // LN, body, 0)
    pltpu.sync_copy(e_v, e_h.at[pl.ds(base, EPW)])
    pltpu.sync_copy(cnt_v, cnt_h.at[pl.ds(pl.multiple_of(w * N, 8), N)])


@functools.cache
def _build_gather_rows():
  return functools.partial(
    pl.kernel,
    out_type=jax.ShapeDtypeStruct((E, H), jnp.float32),
    mesh=_mesh(),
    compiler_params=pltpu.CompilerParams(needs_layout_passes=False),
    scratch_types=[
        pltpu.VMEM((EPW,), jnp.int32),
        pltpu.VMEM((CH, H), jnp.float32),
        pltpu.VMEM((CH, H), jnp.float32),
        pltpu.SemaphoreType.DMA,
        pltpu.SemaphoreType.DMA,
    ],
  )(_gather_rows_body)


def _sc_gather_rows(x, idx):
    return _build_gather_rows()(x, idx)


def _gather_rows_body(x_h, idx_h, out_h, idx_v, rows0_v, rows1_v,
                      sem0, sem1):
    base = _wid() * EPW
    pltpu.sync_copy(idx_h.at[pl.ds(base, EPW)], idx_v)

    def gref(k):
        return x_h.at[idx_v.at[pl.ds(k * CH, CH)]]

    pltpu.async_copy(gref(0), rows0_v, sem0)

    def pair(g, carry):
        k0 = g * 2
        pltpu.async_copy(gref(k0 + 1), rows1_v, sem1)
        pltpu.make_async_copy(gref(k0), rows0_v, sem0).wait()
        pltpu.sync_copy(rows0_v, out_h.at[pl.ds(base + k0 * CH, CH)])
        pltpu.async_copy(gref(k0 + 2), rows0_v, sem0)
        pltpu.make_async_copy(gref(k0 + 1), rows1_v, sem1).wait()
        pltpu.sync_copy(rows1_v, out_h.at[pl.ds(base + (k0 + 1) * CH, CH)])
        return carry

    lax.fori_loop(0, NCHK // 2, pair, 0)
    kl = NCHK - 1
    pltpu.make_async_copy(gref(kl), rows0_v, sem0).wait()
    pltpu.sync_copy(rows0_v, out_h.at[pl.ds(base + kl * CH, CH)])


def _agg_prologue(z_h, shacc):
    s = lax.axis_index("s")
    off = pl.multiple_of(s * NPT, 8)
    pltpu.sync_copy(z_h.at[pl.ds(0, NPT)], shacc.at[pl.ds(off, NPT)])

    @pl.when(s == 0)
    def _zero_tail():
        pltpu.sync_copy(z_h.at[pl.ds(0, NTAIL)],
                        shacc.at[pl.ds(NS * NPT, NTAIL)])

    plsc.subcore_barrier()
    return off


def _agg_epilogue(off, acc_h, shacc):
    c = lax.axis_index("c")
    s = lax.axis_index("s")
    plsc.subcore_barrier()
    pltpu.sync_copy(shacc.at[pl.ds(off, NPT)],
                    acc_h.at[c, pl.ds(off, NPT)])

    @pl.when(s == 0)
    def _dump_tail():
        pltpu.sync_copy(shacc.at[pl.ds(NS * NPT, NTAIL)],
                        acc_h.at[c, pl.ds(NS * NPT, NTAIL)])


def _scale_rows(rows_v, ec_v):
    def srow(r, cc):
        ev = plsc.load_gather(ec_v, [jnp.full((LN,), r, jnp.int32)])
        for j in range(H // LN):
            sl = pl.ds(j * LN, LN)
            rows_v[r, sl] = rows_v[r, sl] * ev
        return cc

    lax.fori_loop(0, CH, srow, 0)


def _agg_ring(gref, e_h, dst_h, base, ec_v, rowsb, dstc, gsems, ssems,
              shacc):
    def abc(k, i):
        pltpu.make_async_copy(gref(k), rowsb[i], gsems[i]).wait()
        pltpu.sync_copy(e_h.at[pl.ds(base + k * CH, CH)], ec_v)
        pltpu.sync_copy(dst_h.at[pl.ds(base + k * CH, CH)], dstc[i])
        _scale_rows(rowsb[i], ec_v)
        pltpu.async_copy(rowsb[i], shacc.at[dstc[i]], ssems[i], add=True)

    def wait_sc(j):
        pltpu.make_async_copy(rowsb[j], shacc.at[dstc[j]], ssems[j]).wait()

    pltpu.async_copy(gref(0), rowsb[0], gsems[0])
    pltpu.async_copy(gref(1), rowsb[1], gsems[1])

    def group(g, carry):
        for i in range(3):
            k = g * 3 + i
            abc(k, i)
            j = (i + 2) % 3

            @pl.when(k >= 1)
            def _drain():
                wait_sc(j)

            pltpu.async_copy(gref(k + 2), rowsb[j], gsems[j])
        return carry

    lax.fori_loop(0, NCHK // 3, group, 0)
    kt = (NCHK // 3) * 3
    for k in range(kt, NCHK):
        i = k % 3
        abc(k, i)
        wait_sc((i + 2) % 3)
    wait_sc((NCHK - 1) % 3)


@functools.cache
def _build_agg_gather():
  return functools.partial(
    pl.kernel,
    out_type=jax.ShapeDtypeStruct((NC, N, H), jnp.float32),
    mesh=_mesh(),
    compiler_params=pltpu.CompilerParams(needs_layout_passes=False),
    scratch_types=[
        pltpu.VMEM((EPW,), jnp.int32),
        pltpu.VMEM((CH,), jnp.float32),
        pltpu.VMEM((CH,), jnp.int32),
        pltpu.VMEM((CH,), jnp.int32),
        pltpu.VMEM((CH,), jnp.int32),
        pltpu.VMEM((CH, H), jnp.float32),
        pltpu.VMEM((CH, H), jnp.float32),
        pltpu.VMEM((CH, H), jnp.float32),
        pltpu.SemaphoreType.DMA,
        pltpu.SemaphoreType.DMA,
        pltpu.SemaphoreType.DMA,
        pltpu.SemaphoreType.DMA,
        pltpu.SemaphoreType.DMA,
        pltpu.SemaphoreType.DMA,
        pltpu.VMEM_SHARED((N, H), jnp.float32),
    ],
  )(_agg_gather_body)


def _sc_agg_gather(x, e, srcf, dstf, z):
    return _build_agg_gather()(x, e, srcf, dstf, z)


def _agg_gather_body(x_h, e_h, src_h, dst_h, z_h, acc_h,
                     src_v, ec_v, d0, d1, d2, r0, r1, r2,
                     g0, g1, g2, s0, s1, s2, shacc):
    w = _wid()
    base = w * EPW
    pltpu.sync_copy(src_h.at[pl.ds(base, EPW)], src_v)
    off = _agg_prologue(z_h, shacc)

    def gref(k):
        return x_h.at[src_v.at[pl.ds(k * CH, CH)]]

    _agg_ring(gref, e_h, dst_h, base, ec_v, (r0, r1, r2), (d0, d1, d2),
              (g0, g1, g2), (s0, s1, s2), shacc)
    _agg_epilogue(off, acc_h, shacc)


@functools.cache
def _build_agg_stream():
  return functools.partial(
    pl.kernel,
    out_type=jax.ShapeDtypeStruct((NC, N, H), jnp.float32),
    mesh=_mesh(),
    compiler_params=pltpu.CompilerParams(needs_layout_passes=False),
    scratch_types=[
        pltpu.VMEM((CH,), jnp.float32),
        pltpu.VMEM((CH,), jnp.int32),
        pltpu.VMEM((CH,), jnp.int32),
        pltpu.VMEM((CH,), jnp.int32),
        pltpu.VMEM((CH, H), jnp.float32),
        pltpu.VMEM((CH, H), jnp.float32),
        pltpu.VMEM((CH, H), jnp.float32),
        pltpu.SemaphoreType.DMA,
        pltpu.SemaphoreType.DMA,
        pltpu.SemaphoreType.DMA,
        pltpu.SemaphoreType.DMA,
        pltpu.SemaphoreType.DMA,
        pltpu.SemaphoreType.DMA,
        pltpu.VMEM_SHARED((N, H), jnp.float32),
    ],
  )(_agg_stream_body)


def _sc_agg_stream(msg, e, dstf, z):
    return _build_agg_stream()(msg, e, dstf, z)


def _agg_stream_body(msg_h, e_h, dst_h, z_h, acc_h,
                     ec_v, d0, d1, d2, r0, r1, r2,
                     g0, g1, g2, s0, s1, s2, shacc):
    w = _wid()
    base = w * EPW
    off = _agg_prologue(z_h, shacc)

    def gref(k):
        return msg_h.at[pl.ds(base + k * CH, CH)]

    _agg_ring(gref, e_h, dst_h, base, ec_v, (r0, r1, r2), (d0, d1, d2),
              (g0, g1, g2), (s0, s1, s2), shacc)
    _agg_epilogue(off, acc_h, shacc)


def _mmT(x, w):
    return lax.dot_general(x, w, (((1,), (1,)), ((), ())),
                           preferred_element_type=jnp.float32)


def _lrelu(v):
    return jnp.where(v > 0, v, 0.01 * v)


def _elu(v):
    return jnp.where(v > 0, v, jnp.exp(jnp.minimum(v, 0.0)) - 1.0)


def _gru_step(h_in, h_state, wih, whh, bih, bhh):
    gi = _mmT(h_in, wih) + bih
    gh = _mmT(h_state, whh) + bhh
    r = jax.nn.sigmoid(gi[:, :H] + gh[:, :H])
    z = jax.nn.sigmoid(gi[:, H:2 * H] + gh[:, H:2 * H])
    n = jnp.tanh(gi[:, 2 * H:] + r * gh[:, 2 * H:])
    return (1.0 - z) * n + z * h_state


def _full(shape):
    return pl.BlockSpec(shape, lambda *_: (0,) * len(shape))


def _tc_pre(raw, l1W, l1b, gnWa, gnb, ga1, gab):
    def body(raw_r, l1W_r, l1b_r, gnWa_r, gnb_r, ga1_r, gab_r,
             x_r, u_r, s1_r, mx_r):
        x = _lrelu(_mmT(raw_r[...], l1W_r[...]) + l1b_r[...])
        x_r[...] = x
        u_r[...] = _mmT(raw_r[...], gnWa_r[...]) + gnb_r[...]
        s1 = jnp.sum(x * ga1_r[...], axis=1, keepdims=True) + gab_r[0, 0]
        s1_r[...] = s1

        @pl.when(pl.program_id(0) == 0)
        def _init():
            mx_r[...] = jnp.full((1, 1), -1e30, jnp.float32)

        mx_r[...] = jnp.maximum(mx_r[...], jnp.max(s1))

    nblk = N // NB
    return pl.pallas_call(
        body,
        grid=(nblk,),
        in_specs=[
            pl.BlockSpec((NB, D), lambda i: (i, 0)),
            _full((H, D)), _full((1, H)), _full((H, D)), _full((1, H)),
            _full((1, H)), _full((1, 1)),
        ],
        out_specs=[
            pl.BlockSpec((NB, H), lambda i: (i, 0)),
            pl.BlockSpec((NB, H), lambda i: (i, 0)),
            pl.BlockSpec((NB, 1), lambda i: (i, 0)),
            pl.BlockSpec((1, 1), lambda i: (0, 0)),
        ],
        out_shape=[
            jax.ShapeDtypeStruct((N, H), jnp.float32),
            jax.ShapeDtypeStruct((N, H), jnp.float32),
            jax.ShapeDtypeStruct((N, 1), jnp.float32),
            jax.ShapeDtypeStruct((1, 1), jnp.float32),
        ],
    )(raw, l1W, l1b, gnWa, gnb, ga1, gab)


def _tc_xj(us, edge_attr, gnWb, ga2):
    def body(us_r, ea_r, w_r, ga2_r, xj_r, t_r, mx_r):
        xj = _lrelu(us_r[...] + _mmT(ea_r[...], w_r[...]))
        xj_r[...] = xj
        t = jnp.sum(xj * ga2_r[...], axis=1, keepdims=True)
        t_r[...] = t

        @pl.when(pl.program_id(0) == 0)
        def _init():
            mx_r[...] = jnp.full((1, 1), -1e30, jnp.float32)

        mx_r[...] = jnp.maximum(mx_r[...], jnp.max(t))

    return pl.pallas_call(
        body,
        grid=(E // EB,),
        in_specs=[
            pl.BlockSpec((EB, H), lambda i: (i, 0)),
            pl.BlockSpec((EB, ED), lambda i: (i, 0)),
            _full((H, ED)),
            _full((1, H)),
        ],
        out_specs=[
            pl.BlockSpec((EB, H), lambda i: (i, 0)),
            pl.BlockSpec((EB, 1), lambda i: (i, 0)),
            pl.BlockSpec((1, 1), lambda i: (0, 0)),
        ],
        out_shape=[
            jax.ShapeDtypeStruct((E, H), jnp.float32),
            jax.ShapeDtypeStruct((E, 1), jnp.float32),
            jax.ShapeDtypeStruct((1, 1), jnp.float32),
        ],
    )(us, edge_attr, gnWb, ga2)


def _tc_red_cnt(cnt):
    def body(cnt_r, s_r):
        ones = jnp.ones((NW, 1), jnp.float32)
        s_r[...] = lax.dot_general(cnt_r[...], ones, (((0,), (0,)), ((), ())),
                                   preferred_element_type=jnp.float32)

    return pl.pallas_call(
        body,
        in_specs=[_full((NW, N))],
        out_specs=_full((N, 1)),
        out_shape=jax.ShapeDtypeStruct((N, 1), jnp.float32),
    )(cnt)


def _tc_post(acc, ssum, x, tW, tb, wih, whh, bih, bhh, a1n, a2n, abn):
    def body(acc_r, ssum_r, x_r, tW_r, tb_r, wih_r, whh_r, bih_r, bhh_r,
             a1_r, a2_r, ab_r, xn_r, s1_r, s2_r, mx1_r, mx2_r):
        ssum = ssum_r[...]
        r = 1.0 / (ssum + 1e-16)
        accs = (acc_r[0] + acc_r[1]) * r
        cnt = ssum * r
        h = _elu(_mmT(accs, tW_r[...]) + cnt * tb_r[...])
        xn = jax.nn.relu(_gru_step(h, x_r[...], wih_r[...], whh_r[...],
                                   bih_r[...], bhh_r[...]))
        xn_r[...] = xn
        s1 = jnp.sum(xn * a1_r[...], axis=1, keepdims=True) + ab_r[0, 0]
        s2 = jnp.sum(xn * a2_r[...], axis=1, keepdims=True)
        s1_r[...] = s1
        s2_r[...] = s2

        @pl.when(pl.program_id(0) == 0)
        def _init():
            mx1_r[...] = jnp.full((1, 1), -1e30, jnp.float32)
            mx2_r[...] = jnp.full((1, 1), -1e30, jnp.float32)

        mx1_r[...] = jnp.maximum(mx1_r[...], jnp.max(s1))
        mx2_r[...] = jnp.maximum(mx2_r[...], jnp.max(s2))

    nblk = N // NB
    return pl.pallas_call(
        body,
        grid=(nblk,),
        in_specs=[
            pl.BlockSpec((NC, NB, H), lambda i: (0, i, 0)),
            pl.BlockSpec((NB, 1), lambda i: (i, 0)),
            pl.BlockSpec((NB, H), lambda i: (i, 0)),
            _full((H, H)), _full((1, H)),
            _full((3 * H, H)), _full((3 * H, H)),
            _full((1, 3 * H)), _full((1, 3 * H)),
            _full((1, H)), _full((1, H)), _full((1, 1)),
        ],
        out_specs=[
            pl.BlockSpec((NB, H), lambda i: (i, 0)),
            pl.BlockSpec((NB, 1), lambda i: (i, 0)),
            pl.BlockSpec((NB, 1), lambda i: (i, 0)),
            pl.BlockSpec((1, 1), lambda i: (0, 0)),
            pl.BlockSpec((1, 1), lambda i: (0, 0)),
        ],
        out_shape=[
            jax.ShapeDtypeStruct((N, H), jnp.float32),
            jax.ShapeDtypeStruct((N, 1), jnp.float32),
            jax.ShapeDtypeStruct((N, 1), jnp.float32),
            jax.ShapeDtypeStruct((1, 1), jnp.float32),
            jax.ShapeDtypeStruct((1, 1), jnp.float32),
        ],
    )(acc, ssum, x, tW, tb, wih, whh, bih, bhh, a1n, a2n, abn)


def _tc_mol(xb, xa, batch2, ma1, ma2, mab, mtW, mtb,
            mwih, mwhh, mbih, mbhh, l2W, l2b):
    NBLK = N // NB
    neg = -1e30

    def body(xb_r, xa_r, b_r, ma1_r, ma2_r, mab_r, mtW_r, mtb_r,
             mwih_r, mwhh_r, mbih_r, mbhh_r, l2W_r, l2b_r, out_r, sc_r):
        iota_g = lax.broadcasted_iota(jnp.int32, (NB, G), 1)

        def maskf(b):
            bb = b_r[pl.ds(b * NB, NB), :]
            return (bb == iota_g).astype(jnp.float32)

        pool = jnp.zeros((G, H), jnp.float32)
        for b in range(NBLK):
            mf = maskf(b)
            pool = pool + lax.dot_general(
                mf, xb_r[pl.ds(b * NB, NB), :], (((0,), (0,)), ((), ())),
                preferred_element_type=jnp.float32)
        out = jax.nn.relu(pool)

        for t in range(2):
            def xm(b):
                xbb = xb_r[pl.ds(b * NB, NB), :]
                if t == 0:
                    return (xbb + xa_r[pl.ds(b * NB, NB), :]) * 0.5
                return xbb

            s1row = _mmT(ma1_r[...], out) + mab_r[0, 0]
            m = jnp.full((1, G), neg, jnp.float32)
            for b in range(NBLK):
                mf = maskf(b)
                s2m = jnp.sum(xm(b) * ma2_r[...], axis=1,
                              keepdims=True)
                g1 = jnp.sum(mf * s1row, axis=1, keepdims=True)
                sc = _lrelu(g1 + s2m)
                sc_r[pl.ds(b * NB, NB), :] = sc
                mw = jnp.where(mf > 0, sc, neg)
                m = jnp.maximum(m, jnp.max(mw, axis=0, keepdims=True))
            m = jnp.where(m > neg * 0.5, m, 0.0)
            ssum = jnp.zeros((1, G), jnp.float32)
            for b in range(NBLK):
                mf = maskf(b)
                mg = jnp.sum(mf * m, axis=1, keepdims=True)
                e = jnp.exp(sc_r[pl.ds(b * NB, NB), :] - mg)
                sc_r[pl.ds(b * NB, NB), :] = e
                ssum = ssum + lax.dot_general(
                    e, mf, (((0,), (0,)), ((), ())),
                    preferred_element_type=jnp.float32)
            accm = jnp.zeros((G, H), jnp.float32)
            cntc = jnp.zeros((G, 1), jnp.float32)
            for b in range(NBLK):
                mf = maskf(b)
                denom = jnp.sum(mf * ssum, axis=1, keepdims=True)
                alpha = sc_r[pl.ds(b * NB, NB), :] / (denom + 1e-16)
                accm = accm + lax.dot_general(
                    mf, alpha * xm(b), (((0,), (0,)), ((), ())),
                    preferred_element_type=jnp.float32)
                cntc = cntc + lax.dot_general(
                    mf, alpha, (((0,), (0,)), ((), ())),
                    preferred_element_type=jnp.float32)
            h = _elu(_mmT(accm, mtW_r[...]) + cntc * mtb_r[...])
            out = jax.nn.relu(_gru_step(h, out, mwih_r[...], mwhh_r[...],
                                        mbih_r[...], mbhh_r[...]))

        out_r[...] = _mmT(out, l2W_r[...]) + l2b_r[...]

    return pl.pallas_call(
        body,
        in_specs=[
            _full((N, H)), _full((N, H)), _full((N, 1)),
            _full((1, H)), _full((1, H)), _full((1, 1)),
            _full((H, H)), _full((1, H)),
            _full((3 * H, H)), _full((3 * H, H)),
            _full((1, 3 * H)), _full((1, 3 * H)),
            _full((OUT, H)), _full((1, OUT)),
        ],
        out_specs=_full((G, OUT)),
        out_shape=jax.ShapeDtypeStruct((G, OUT), jnp.float32),
        scratch_shapes=[pltpu.VMEM((N, 1), jnp.float32)],
    )(xb, xa, batch2, ma1, ma2, mab, mtW, mtb,
      mwih, mwhh, mbih, mbhh, l2W, l2b)


def _row(v):
    return v.reshape(1, -1)


def _b16(m):
    return jnp.broadcast_to(m.reshape(1), (LN,))


def kernel(raw, edge_index, edge_attr, batch, params):
    p = params
    src = edge_index[0]
    dst = edge_index[1]
    zrows = jnp.zeros((NPT, H), jnp.float32)

    ga1 = _row(p['gate_align_W'][0, :H])
    ga2 = _row(p['gate_align_W'][0, H:])
    gab = p['gate_align_b'].reshape(1, 1)
    x0, u, s1g, mx1g = _tc_pre(raw, p['lin1_W'], _row(p['lin1_b']),
                               p['gate_nei_W'][:, :D], _row(p['gate_nei_b']),
                               ga1, gab)
    us = _sc_gather_rows(u, src)
    xj, t, mxt = _tc_xj(us, edge_attr, p['gate_nei_W'][:, D:], ga2)

    ef, cnt = _sc_escore_gate(s1g.reshape(N), t.reshape(E), dst,
                              _b16(mx1g), _b16(mxt))
    acc = _sc_agg_stream(xj, ef, dst, zrows)
    ssum = _tc_red_cnt(cnt.reshape(NW, N))
    c1a1 = _row(p['conv1_align_W'][0, :H])
    c1a2 = _row(p['conv1_align_W'][0, H:])
    x1, s1, s2, mx1, mx2 = _tc_post(
        acc, ssum, x0,
        p['gate_attend_W'], _row(p['gate_attend_b']),
        p['gru0_wih'], p['gru0_whh'],
        _row(p['gru0_bih']), _row(p['gru0_bhh']),
        c1a1, c1a2, p['conv1_align_b'].reshape(1, 1))

    xs_in = x1
    for i in (1, 2):
        ef, cnt = _sc_escore_conv(s1.reshape(N), s2.reshape(N), src, dst,
                                  _b16(mx1), _b16(mx2))
        acc = _sc_agg_gather(xs_in, ef, src, dst, zrows)
        ssum = _tc_red_cnt(cnt.reshape(NW, N))
        if i == 1:
            na1 = _row(p['conv2_align_W'][0, :H])
            na2 = _row(p['conv2_align_W'][0, H:])
            nab = p['conv2_align_b'].reshape(1, 1)
        else:
            na1, na2, nab = c1a1, c1a2, p['conv1_align_b'].reshape(1, 1)
        xs_in, s1, s2, mx1, mx2 = _tc_post(
            acc, ssum, xs_in,
            p['conv%d_attend_W' % i], _row(p['conv%d_attend_b' % i]),
            p['gru%d_wih' % i], p['gru%d_whh' % i],
            _row(p['gru%d_bih' % i]), _row(p['gru%d_bhh' % i]),
            na1, na2, nab)
        if i == 1:
            xa = xs_in
    xb = xs_in

    return _tc_mol(xb, xa, batch.reshape(N, 1),
                   _row(p['mol_align_W'][0, :H]), _row(p['mol_align_W'][0, H:]),
                   p['mol_align_b'].reshape(1, 1),
                   p['mol_attend_W'], _row(p['mol_attend_b']),
                   p['mgru_wih'], p['mgru_whh'],
                   _row(p['mgru_bih']), _row(p['mgru_bhh']),
                   p['lin2_W'], _row(p['lin2_b']))
